# Initial kernel scaffold; baseline (speedup 1.0000x reference)
import jax
import jax.numpy as jnp
from jax.experimental import pallas as pl


def kernel(x, Wr, br, W1, b1, W2, b2):
    raise NotImplementedError("write your pallas kernel here")


# speedup vs baseline: 2.0128x; 2.0128x over previous
import functools

import jax
import jax.numpy as jnp
from jax.experimental import pallas as pl
from jax.experimental.pallas import tpu as pltpu

EMB = 1024
HID = 1536
NEXP = 8
TOPK = 2

TOK_TILE = 256
LANE_PAD = 128

_INV_SQRT2 = 0.7071067811865476


def _moe_dense_body(x_ref, wr_ref, br_ref, w1_ref, b1_ref, w2_ref, b2_ref,
                    out_ref, wscr, acc):
    e = pl.program_id(0)
    t = pl.program_id(1)
    xt = x_ref[...]

    @pl.when(e == 0)
    def _router():
        logits = jax.lax.dot_general(
            xt, wr_ref[...], (((1,), (1,)), ((), ())),
            preferred_element_type=jnp.float32)
        logits = logits + br_ref[0:1, :]
        lane = jax.lax.broadcasted_iota(jnp.int32, (TOK_TILE, LANE_PAD), 1)
        logits = jnp.where(lane < NEXP, logits, jnp.float32(-1e30))
        m = jnp.max(logits, axis=1, keepdims=True)
        p = jnp.exp(logits - m)
        p = p / jnp.sum(p, axis=1, keepdims=True)
        m1 = jnp.max(p, axis=1, keepdims=True)
        p2 = jnp.where(p >= m1, jnp.float32(-1.0), p)
        m2 = jnp.max(p2, axis=1, keepdims=True)
        w = jnp.where(p >= m2, p, jnp.float32(0.0))
        wscr[pl.ds(t * TOK_TILE, TOK_TILE), :] = w

    w_tile = wscr[pl.ds(t * TOK_TILE, TOK_TILE), :]
    lane = jax.lax.broadcasted_iota(jnp.int32, (TOK_TILE, LANE_PAD), 1)
    we = jnp.sum(jnp.where(lane == e, w_tile, 0.0), axis=1, keepdims=True)

    h = jax.lax.dot_general(
        xt, w1_ref[0], (((1,), (1,)), ((), ())),
        preferred_element_type=jnp.float32)
    h = h + b1_ref[0]
    g = 0.5 * h * (1.0 + jax.lax.erf(h * _INV_SQRT2))
    eo = jax.lax.dot_general(
        g, w2_ref[0], (((1,), (1,)), ((), ())),
        preferred_element_type=jnp.float32)
    eo = eo + b2_ref[0]
    contrib = eo * we
    rows = pl.ds(t * TOK_TILE, TOK_TILE)

    @pl.when(e == 0)
    def _init():
        acc[rows, :] = contrib

    @pl.when(e != 0)
    def _acc():
        acc[rows, :] = acc[rows, :] + contrib

    @pl.when(e == NEXP - 1)
    def _flush():
        out_ref[...] = acc[rows, :]


def kernel(x, Wr, br, W1, b1, W2, b2):
    B, N, E = x.shape
    T = B * N
    x2 = x.reshape(T, E)
    n_tiles = T // TOK_TILE

    wr_pad = jnp.zeros((LANE_PAD, E), jnp.float32).at[:NEXP].set(Wr)
    br_pad = jnp.zeros((8, LANE_PAD), jnp.float32).at[:, :NEXP].set(br[None, :])

    out = pl.pallas_call(
        _moe_dense_body,
        grid=(NEXP, n_tiles),
        in_specs=[
            pl.BlockSpec((TOK_TILE, E), lambda e, t: (t, 0)),
            pl.BlockSpec((LANE_PAD, E), lambda e, t: (0, 0)),
            pl.BlockSpec((8, LANE_PAD), lambda e, t: (0, 0)),
            pl.BlockSpec((1, HID, E), lambda e, t: (e, 0, 0)),
            pl.BlockSpec((1, 1, HID), lambda e, t: (e, 0, 0)),
            pl.BlockSpec((1, E, HID), lambda e, t: (e, 0, 0)),
            pl.BlockSpec((1, 1, E), lambda e, t: (e, 0, 0)),
        ],
        out_specs=pl.BlockSpec((TOK_TILE, E), lambda e, t: (t, 0)),
        out_shape=jax.ShapeDtypeStruct((T, E), jnp.float32),
        scratch_shapes=[pltpu.VMEM((T, LANE_PAD), jnp.float32),
                        pltpu.VMEM((T, E), jnp.float32)],
        compiler_params=pltpu.CompilerParams(
            dimension_semantics=("arbitrary", "arbitrary")),
    )(x2, wr_pad, br_pad, W1, b1.reshape(NEXP, 1, HID), W2,
      b2.reshape(NEXP, 1, E))
    return out.reshape(B, N, E)
